# baseline (speedup 1.0000x reference)
import functools

import jax
import jax.numpy as jnp
import numpy as np
from jax.experimental import pallas as pl
from jax.experimental.pallas import tpu as pltpu

_TILE = 408


def _fold_kernel(cwt_ref, wd_ref, cb_ref, bd_ref, w2_ref, bias_ref):
    wd = wd_ref[...]
    for t in range(3):
        w2_ref[t] = jax.lax.dot_general(
            cwt_ref[t], wd, (((0,), (0,)), ((), ())),
            preferred_element_type=jnp.float32
        ).astype(jnp.bfloat16)
    bias_ref[...] = (
        4.0 * jnp.dot(cb_ref[...], wd, preferred_element_type=jnp.float32)
        + bd_ref[...]
    )


def _main_kernel(xc_ref, ph_ref, nh_ref, w2_ref, q_ref, bias_ref, o_ref,
                 scr_ref, *, tile, n_tiles, length, nb):
    lt = pl.program_id(1)

    w0 = w2_ref[0]
    w1 = w2_ref[1]
    w2v = w2_ref[2]
    qv = q_ref[...]
    bias = bias_ref[...]
    for b in range(nb):
        xc = xc_ref[b]
        prev = jnp.where(lt == 0, 0.0, ph_ref[b, 7:8, :])
        nxt = jnp.where(lt == n_tiles - 1, 0.0, nh_ref[b, 0:1, :])
        xwin = jnp.concatenate([prev, xc, nxt], axis=0).astype(jnp.bfloat16)

        d0 = jnp.dot(xwin, w0, preferred_element_type=jnp.float32)
        d1 = jnp.dot(xwin, w1, preferred_element_type=jnp.float32)
        d2 = jnp.dot(xwin, w2v, preferred_element_type=jnp.float32)
        scr_ref[b, 0, 0:tile + 2, :] = d1
        scr_ref[b, 1, 0:tile + 2, :] = d2
        a = (d0[0:tile]
             + scr_ref[b, 0, pl.ds(1, tile), :]
             + scr_ref[b, 1, pl.ds(2, tile), :])
        s = jnp.dot(qv, a.astype(jnp.bfloat16),
                    preferred_element_type=jnp.float32)
        o_ref[b] = s + bias


@functools.lru_cache(maxsize=None)
def _pool_matrix(tile):
    q = np.zeros((tile, tile), np.float32)
    for b in (1, 2, 3, 4):
        q += np.kron(np.eye(tile // b, dtype=np.float32),
                     np.full((b, b), 1.0 / b, np.float32))
    return q


def kernel(X, conv_w, conv_b, wr, br, wd, bd):
    n, length, d = X.shape
    del wr, br
    tile = _TILE
    n_tiles = length // tile

    cwt = conv_w.transpose(2, 0, 1)
    w2, bias = pl.pallas_call(
        _fold_kernel,
        out_shape=(
            jax.ShapeDtypeStruct((3, d, d), jnp.bfloat16),
            jax.ShapeDtypeStruct((1, d), jnp.float32),
        ),
        name="gbst_fold",
    )(cwt, wd, conv_b.reshape(1, d), bd.reshape(1, d))

    q = jnp.asarray(_pool_matrix(tile), dtype=jnp.bfloat16)

    nb = 4
    body = functools.partial(
        _main_kernel, tile=tile, n_tiles=n_tiles, length=length, nb=nb)
    out = pl.pallas_call(
        body,
        grid=(n // nb, n_tiles),
        in_specs=[
            pl.BlockSpec((nb, tile, d), lambda i, j: (i, j, 0)),
            pl.BlockSpec(
                (nb, 8, d),
                lambda i, j: (i, jnp.maximum(j * (tile // 8) - 1, 0), 0)),
            pl.BlockSpec(
                (nb, 8, d),
                lambda i, j: (i, jnp.minimum((j + 1) * (tile // 8),
                                             length // 8 - 1), 0)),
            pl.BlockSpec((3, d, d), lambda i, j: (0, 0, 0)),
            pl.BlockSpec((tile, tile), lambda i, j: (0, 0)),
            pl.BlockSpec((1, d), lambda i, j: (0, 0)),
        ],
        out_specs=pl.BlockSpec((nb, tile, d), lambda i, j: (i, j, 0)),
        out_shape=jax.ShapeDtypeStruct((n, length, d), jnp.float32),
        scratch_shapes=[pltpu.VMEM((nb, 2, tile + 8, d), jnp.float32)],
        compiler_params=pltpu.CompilerParams(
            dimension_semantics=("parallel", "arbitrary"),
            vmem_limit_bytes=50 * 1024 * 1024,
        ),
        name="gbst_main",
    )(X, X, X, w2, q, bias)
    return out


# speedup vs baseline: 1.0449x; 1.0449x over previous
import functools

import jax
import jax.numpy as jnp
import numpy as np
from jax.experimental import pallas as pl
from jax.experimental.pallas import tpu as pltpu

_TILE = 408


def _fold_kernel(cwt_ref, wd_ref, cb_ref, bd_ref, w2_ref, bias_ref):
    wd = wd_ref[...]
    for t in range(3):
        w2_ref[t] = jax.lax.dot_general(
            cwt_ref[t], wd, (((0,), (0,)), ((), ())),
            preferred_element_type=jnp.float32
        ).astype(jnp.bfloat16)
    bias_ref[...] = (
        4.0 * jnp.dot(cb_ref[...], wd, preferred_element_type=jnp.float32)
        + bd_ref[...]
    )


def _main_kernel(xc_ref, ph_ref, nh_ref, w2_ref, q_ref, bias_ref, o_ref,
                 scr_ref, *, tile, n_tiles, length, nb):
    lt = pl.program_id(1)

    w0 = w2_ref[0]
    w1 = w2_ref[1]
    w2v = w2_ref[2]
    qv = q_ref[...]
    bias = bias_ref[...]
    for b in range(nb):
        xc = xc_ref[b]
        prev = jnp.where(lt == 0, 0.0, ph_ref[b, 7:8, :])
        nxt = jnp.where(lt == n_tiles - 1, 0.0, nh_ref[b, 0:1, :])
        xwin = jnp.concatenate([prev, xc, nxt], axis=0).astype(jnp.bfloat16)

        d0 = jnp.dot(xwin, w0, preferred_element_type=jnp.float32)
        d1 = jnp.dot(xwin, w1, preferred_element_type=jnp.float32)
        d2 = jnp.dot(xwin, w2v, preferred_element_type=jnp.float32)
        scr_ref[b, 0, 0:tile + 2, :] = d1
        scr_ref[b, 1, 0:tile + 2, :] = d2
        a = (d0[0:tile]
             + scr_ref[b, 0, pl.ds(1, tile), :]
             + scr_ref[b, 1, pl.ds(2, tile), :])
        s = jnp.dot(qv, a.astype(jnp.bfloat16),
                    preferred_element_type=jnp.float32)
        o_ref[b] = s + bias


@functools.lru_cache(maxsize=None)
def _pool_matrix(tile):
    q = np.zeros((tile, tile), np.float32)
    for b in (1, 2, 3, 4):
        q += np.kron(np.eye(tile // b, dtype=np.float32),
                     np.full((b, b), 1.0 / b, np.float32))
    return q


def kernel(X, conv_w, conv_b, wr, br, wd, bd):
    n, length, d = X.shape
    del wr, br
    tile = _TILE
    n_tiles = length // tile

    cwt = conv_w.transpose(2, 0, 1)
    w2, bias = pl.pallas_call(
        _fold_kernel,
        out_shape=(
            jax.ShapeDtypeStruct((3, d, d), jnp.bfloat16),
            jax.ShapeDtypeStruct((1, d), jnp.float32),
        ),
        compiler_params=pltpu.CompilerParams(
            allow_input_fusion=[True, False, False, False],
        ),
        name="gbst_fold",
    )(cwt, wd, conv_b.reshape(1, d), bd.reshape(1, d))

    q = jnp.asarray(_pool_matrix(tile), dtype=jnp.bfloat16)

    nb = 4
    body = functools.partial(
        _main_kernel, tile=tile, n_tiles=n_tiles, length=length, nb=nb)
    out = pl.pallas_call(
        body,
        grid=(n // nb, n_tiles),
        in_specs=[
            pl.BlockSpec((nb, tile, d), lambda i, j: (i, j, 0)),
            pl.BlockSpec(
                (nb, 8, d),
                lambda i, j: (i, jnp.maximum(j * (tile // 8) - 1, 0), 0)),
            pl.BlockSpec(
                (nb, 8, d),
                lambda i, j: (i, jnp.minimum((j + 1) * (tile // 8),
                                             length // 8 - 1), 0)),
            pl.BlockSpec((3, d, d), lambda i, j: (0, 0, 0)),
            pl.BlockSpec((tile, tile), lambda i, j: (0, 0)),
            pl.BlockSpec((1, d), lambda i, j: (0, 0)),
        ],
        out_specs=pl.BlockSpec((nb, tile, d), lambda i, j: (i, j, 0)),
        out_shape=jax.ShapeDtypeStruct((n, length, d), jnp.float32),
        scratch_shapes=[pltpu.VMEM((nb, 2, tile + 8, d), jnp.float32)],
        compiler_params=pltpu.CompilerParams(
            dimension_semantics=("parallel", "arbitrary"),
            vmem_limit_bytes=50 * 1024 * 1024,
        ),
        name="gbst_main",
    )(X, X, X, w2, q, bias)
    return out
